# Initial kernel scaffold; baseline (speedup 1.0000x reference)
import functools

import jax
import jax.numpy as jnp
from jax import lax
from jax.experimental import pallas as pl
from jax.experimental.pallas import tpu as pltpu
from jax.experimental.pallas import tpu_sc as plsc

N_ITEMS = 1000000
N_RANKS = 20
BATCH = 16384
TOTAL = BATCH * N_RANKS

_info = plsc.get_sparse_core_info()
NC = _info.num_cores
NS = _info.num_subcores
NW = NC * NS
L = 16

PER_W = TOTAL // NW
GROUPS = PER_W // 80

_mesh = plsc.VectorSubcoreMesh(core_axis_name="c", subcore_axis_name="s")


def _sigmoid(v):
    return 1.0 / (1.0 + jnp.exp(-v))


@functools.partial(
    pl.kernel,
    mesh=_mesh,
    out_type=jax.ShapeDtypeStruct((TOTAL,), jnp.float32),
    scratch_types=[
        pltpu.VMEM((PER_W,), jnp.int32),
        pltpu.VMEM((PER_W,), jnp.float32),
        pltpu.VMEM((32,), jnp.float32),
        pltpu.SemaphoreType.DMA,
    ],
)
def _pbm_kernel(x_hbm, exam_hbm, rel_hbm, out_hbm, idx_v, vals_v, exam_v, sem):
    wid = lax.axis_index("s") * NC + lax.axis_index("c")
    base = wid * PER_W

    pltpu.sync_copy(x_hbm.at[pl.ds(base, PER_W)], idx_v)
    pltpu.sync_copy(exam_hbm, exam_v)

    pltpu.async_copy(rel_hbm.at[idx_v], vals_v, sem).wait()

    iota = lax.iota(jnp.int32, L)
    exam_vecs = []
    for k in range(5):
        phase_idx = (iota + (16 * k)) % N_RANKS
        ev = plsc.load_gather(exam_v, [phase_idx])
        exam_vecs.append(_sigmoid(ev))

    def group_body(g, _):
        b = g * 80
        for k in range(5):
            v = vals_v[pl.ds(b + 16 * k, L)]
            vals_v[pl.ds(b + 16 * k, L)] = _sigmoid(v) * exam_vecs[k]
        return 0

    lax.fori_loop(0, GROUPS, group_body, 0)

    pltpu.sync_copy(vals_v, out_hbm.at[pl.ds(base, PER_W)])


def kernel(x, exam_table, rel_table):
    xf = x.reshape(TOTAL)
    exam = jnp.pad(exam_table.reshape(N_RANKS), (0, 32 - N_RANKS))
    rel = rel_table.reshape(N_ITEMS)
    out = _pbm_kernel(xf, exam, rel)
    return out.reshape(BATCH, N_RANKS)


# speedup vs baseline: 1.0452x; 1.0452x over previous
import functools

import jax
import jax.numpy as jnp
from jax import lax
from jax.experimental import pallas as pl
from jax.experimental.pallas import tpu as pltpu
from jax.experimental.pallas import tpu_sc as plsc

N_ITEMS = 1000000
N_RANKS = 20
BATCH = 16384
TOTAL = BATCH * N_RANKS

_info = plsc.get_sparse_core_info()
NC = _info.num_cores
NS = _info.num_subcores
NW = NC * NS
L = 16

PER_W = TOTAL // NW
GROUPS = PER_W // 80

_mesh = plsc.VectorSubcoreMesh(core_axis_name="c", subcore_axis_name="s")


def _sigmoid(v):
    return 1.0 / (1.0 + jnp.exp(-v))


@functools.partial(
    pl.kernel,
    mesh=_mesh,
    out_type=jax.ShapeDtypeStruct((TOTAL,), jnp.float32),
    scratch_types=[
        pltpu.VMEM((PER_W,), jnp.int32),
        pltpu.VMEM((PER_W,), jnp.float32),
        pltpu.VMEM((80,), jnp.float32),
        pltpu.SemaphoreType.DMA,
    ],
)
def _pbm_kernel(x_hbm, exam_hbm, rel_hbm, out_hbm, idx_v, vals_v, exam_v, sem):
    wid = lax.axis_index("s") * NC + lax.axis_index("c")
    base = wid * PER_W

    pltpu.sync_copy(x_hbm.at[pl.ds(base, PER_W)], idx_v)
    pltpu.sync_copy(exam_hbm, exam_v)

    pltpu.async_copy(rel_hbm.at[idx_v], vals_v, sem).wait()

    exam_vecs = [_sigmoid(exam_v[pl.ds(16 * k, L)]) for k in range(5)]

    def group_body(g, _):
        b = g * 80
        for k in range(5):
            v = vals_v[pl.ds(b + 16 * k, L)]
            vals_v[pl.ds(b + 16 * k, L)] = _sigmoid(v) * exam_vecs[k]
        return 0

    lax.fori_loop(0, GROUPS, group_body, 0)

    pltpu.sync_copy(vals_v, out_hbm.at[pl.ds(base, PER_W)])


def kernel(x, exam_table, rel_table):
    xf = x.reshape(TOTAL)
    exam = jnp.tile(exam_table.reshape(N_RANKS), 4)
    rel = rel_table.reshape(N_ITEMS)
    out = _pbm_kernel(xf, exam, rel)
    return out.reshape(BATCH, N_RANKS)


# speedup vs baseline: 1.0650x; 1.0190x over previous
import functools

import jax
import jax.numpy as jnp
from jax import lax
from jax.experimental import pallas as pl
from jax.experimental.pallas import tpu as pltpu
from jax.experimental.pallas import tpu_sc as plsc

N_ITEMS = 1000000
N_RANKS = 20
BATCH = 16384
TOTAL = BATCH * N_RANKS

_info = plsc.get_sparse_core_info()
NC = _info.num_cores
NS = _info.num_subcores
NW = NC * NS
L = 16

PER_W = TOTAL // NW
ROWS = BATCH // NW

_mesh = plsc.VectorSubcoreMesh(core_axis_name="c", subcore_axis_name="s")


def _sigmoid(v):
    return 1.0 / (1.0 + jnp.exp(-v))


@functools.partial(
    pl.kernel,
    mesh=_mesh,
    out_type=jax.ShapeDtypeStruct((BATCH, 128), jnp.float32),
    scratch_types=[
        pltpu.VMEM((PER_W,), jnp.int32),
        pltpu.VMEM((PER_W + L,), jnp.float32),
        pltpu.VMEM((ROWS, 128), jnp.float32),
        pltpu.VMEM((32,), jnp.float32),
        pltpu.SemaphoreType.DMA,
    ],
)
def _pbm_kernel(x_hbm, exam_hbm, rel_hbm, out_hbm, idx_v, vals_v, resv, exam_v, sem):
    wid = lax.axis_index("s") * NC + lax.axis_index("c")
    base = wid * PER_W
    base_row = wid * ROWS

    pltpu.sync_copy(x_hbm.at[pl.ds(base, PER_W)], idx_v)
    pltpu.sync_copy(exam_hbm, exam_v)

    pltpu.async_copy(rel_hbm.at[idx_v], vals_v.at[pl.ds(0, PER_W)], sem).wait()

    e0 = _sigmoid(exam_v[pl.ds(0, L)])
    e1 = _sigmoid(exam_v[pl.ds(L, L)])

    def row_body(r, _):
        b = r * N_RANKS
        v0 = vals_v[pl.ds(b, L)]
        v1 = vals_v[pl.ds(b + L, L)]
        resv[r, pl.ds(0, L)] = e0 * _sigmoid(v0)
        resv[r, pl.ds(L, L)] = e1 * _sigmoid(v1)
        return 0

    lax.fori_loop(0, ROWS, row_body, 0)

    pltpu.sync_copy(resv, out_hbm.at[pl.ds(base_row, ROWS)])


def kernel(x, exam_table, rel_table):
    xf = x.reshape(TOTAL)
    exam = jnp.pad(exam_table.reshape(N_RANKS), (0, 32 - N_RANKS))
    rel = rel_table.reshape(N_ITEMS)
    out = _pbm_kernel(xf, exam, rel)
    return out[:, :N_RANKS]


# speedup vs baseline: 1.1214x; 1.0529x over previous
import functools

import jax
import jax.numpy as jnp
from jax import lax
from jax.experimental import pallas as pl
from jax.experimental.pallas import tpu as pltpu
from jax.experimental.pallas import tpu_sc as plsc

N_ITEMS = 1000000
N_RANKS = 20
BATCH = 16384
TOTAL = BATCH * N_RANKS

_info = plsc.get_sparse_core_info()
NC = _info.num_cores
NS = _info.num_subcores
NW = NC * NS
L = 16

PER_W = TOTAL // NW
ROWS = BATCH // NW
NCH = 4
CROWS = ROWS // NCH
CVALS = PER_W // NCH

_mesh = plsc.VectorSubcoreMesh(core_axis_name="c", subcore_axis_name="s")


def _sigmoid(v):
    return 1.0 / (1.0 + jnp.exp(-v))


@functools.partial(
    pl.kernel,
    mesh=_mesh,
    out_type=jax.ShapeDtypeStruct((BATCH, 128), jnp.float32),
    scratch_types=[
        pltpu.VMEM((PER_W,), jnp.int32),
        pltpu.VMEM((CVALS + L,), jnp.float32),
        pltpu.VMEM((CVALS + L,), jnp.float32),
        pltpu.VMEM((ROWS, 128), jnp.float32),
        pltpu.VMEM((32,), jnp.float32),
        pltpu.SemaphoreType.DMA,
        pltpu.SemaphoreType.DMA,
        pltpu.SemaphoreType.DMA,
    ],
)
def _pbm_kernel(x_hbm, exam_hbm, rel_hbm, out_hbm,
                idx_v, vals_a, vals_b, resv, exam_v, gsem_a, gsem_b, osem):
    wid = lax.axis_index("s") * NC + lax.axis_index("c")
    base = wid * PER_W
    base_row = wid * ROWS

    vals = (vals_a, vals_b)
    gsem = (gsem_a, gsem_b)

    pltpu.sync_copy(x_hbm.at[pl.ds(base, PER_W)], idx_v)
    pltpu.sync_copy(exam_hbm, exam_v)

    e0 = _sigmoid(exam_v[pl.ds(0, L)])
    e1 = _sigmoid(exam_v[pl.ds(L, L)])

    def start_gather(c):
        return pltpu.async_copy(
            rel_hbm.at[idx_v.at[pl.ds(c * CVALS, CVALS)]],
            vals[c % 2].at[pl.ds(0, CVALS)],
            gsem[c % 2],
        )

    out_copies = []
    gathers = [start_gather(0)]
    for c in range(NCH):
        gathers[c].wait()
        if c + 1 < NCH:
            gathers.append(start_gather(c + 1))
        buf = vals[c % 2]

        def row_body(r, _, buf=buf, c=c):
            b = r * N_RANKS
            v0 = buf[pl.ds(b, L)]
            v1 = buf[pl.ds(b + L, L)]
            row = c * CROWS + r
            resv[row, pl.ds(0, L)] = e0 * _sigmoid(v0)
            resv[row, pl.ds(L, L)] = e1 * _sigmoid(v1)
            return 0

        lax.fori_loop(0, CROWS, row_body, 0)
        out_copies.append(
            pltpu.async_copy(
                resv.at[pl.ds(c * CROWS, CROWS)],
                out_hbm.at[pl.ds(base_row + c * CROWS, CROWS)],
                osem,
            )
        )

    for oc in out_copies:
        oc.wait()


def kernel(x, exam_table, rel_table):
    xf = x.reshape(TOTAL)
    exam = jnp.pad(exam_table.reshape(N_RANKS), (0, 32 - N_RANKS))
    rel = rel_table.reshape(N_ITEMS)
    out = _pbm_kernel(xf, exam, rel)
    return out[:, :N_RANKS]


# speedup vs baseline: 1.1223x; 1.0007x over previous
import functools

import jax
import jax.numpy as jnp
from jax import lax
from jax.experimental import pallas as pl
from jax.experimental.pallas import tpu as pltpu
from jax.experimental.pallas import tpu_sc as plsc

N_ITEMS = 1000000
N_RANKS = 20
BATCH = 16384
TOTAL = BATCH * N_RANKS

_info = plsc.get_sparse_core_info()
NC = _info.num_cores
NS = _info.num_subcores
NW = NC * NS
L = 16

PER_W = TOTAL // NW
ROWS = BATCH // NW
NCH = 4
CROWS = ROWS // NCH
CVALS = PER_W // NCH

_mesh = plsc.VectorSubcoreMesh(core_axis_name="c", subcore_axis_name="s")


def _sigmoid(v):
    return 1.0 / (1.0 + jnp.exp(-v))


@functools.partial(
    pl.kernel,
    mesh=_mesh,
    out_type=jax.ShapeDtypeStruct((BATCH, N_RANKS), jnp.float32),
    scratch_types=[
        pltpu.VMEM((PER_W,), jnp.int32),
        pltpu.VMEM((CVALS + L,), jnp.float32),
        pltpu.VMEM((CVALS + L,), jnp.float32),
        pltpu.VMEM((ROWS, N_RANKS), jnp.float32),
        pltpu.VMEM((32,), jnp.float32),
        pltpu.SemaphoreType.DMA,
        pltpu.SemaphoreType.DMA,
        pltpu.SemaphoreType.DMA,
    ],
)
def _pbm_kernel(x_hbm, exam_hbm, rel_hbm, out_hbm,
                idx_v, vals_a, vals_b, resv, exam_v, gsem_a, gsem_b, osem):
    wid = lax.axis_index("s") * NC + lax.axis_index("c")
    base = wid * PER_W
    base_row = wid * ROWS

    vals = (vals_a, vals_b)
    gsem = (gsem_a, gsem_b)

    pltpu.sync_copy(x_hbm.at[pl.ds(base, PER_W)], idx_v)
    pltpu.sync_copy(exam_hbm, exam_v)

    e0 = _sigmoid(exam_v[pl.ds(0, L)])
    e1 = _sigmoid(exam_v[pl.ds(4, L)])

    def start_gather(c):
        return pltpu.async_copy(
            rel_hbm.at[idx_v.at[pl.ds(c * CVALS, CVALS)]],
            vals[c % 2].at[pl.ds(0, CVALS)],
            gsem[c % 2],
        )

    out_copies = []
    gathers = [start_gather(0)]
    for c in range(NCH):
        gathers[c].wait()
        if c + 1 < NCH:
            gathers.append(start_gather(c + 1))
        buf = vals[c % 2]

        def row_body(r, _, buf=buf, c=c):
            b = r * N_RANKS
            v0 = buf[pl.ds(b, L)]
            v1 = buf[pl.ds(b + 4, L)]
            row = c * CROWS + r
            resv[row, pl.ds(0, L)] = e0 * _sigmoid(v0)
            resv[row, pl.ds(4, L)] = e1 * _sigmoid(v1)
            return 0

        lax.fori_loop(0, CROWS, row_body, 0)
        out_copies.append(
            pltpu.async_copy(
                resv.at[pl.ds(c * CROWS, CROWS)],
                out_hbm.at[pl.ds(base_row + c * CROWS, CROWS)],
                osem,
            )
        )

    for oc in out_copies:
        oc.wait()


def kernel(x, exam_table, rel_table):
    xf = x.reshape(TOTAL)
    exam = jnp.pad(exam_table.reshape(N_RANKS), (0, 32 - N_RANKS))
    rel = rel_table.reshape(N_ITEMS)
    return _pbm_kernel(xf, exam, rel)


# speedup vs baseline: 1.1475x; 1.0225x over previous
import functools

import jax
import jax.numpy as jnp
from jax import lax
from jax.experimental import pallas as pl
from jax.experimental.pallas import tpu as pltpu
from jax.experimental.pallas import tpu_sc as plsc

N_ITEMS = 1000000
N_RANKS = 20
BATCH = 16384
TOTAL = BATCH * N_RANKS

_info = plsc.get_sparse_core_info()
NC = _info.num_cores
NS = _info.num_subcores
NW = NC * NS
L = 16

PER_W = TOTAL // NW
ROWS = BATCH // NW
NCH = 8
NBUF = 3
CROWS = ROWS // NCH
CVALS = PER_W // NCH
UNROLL = 4

_mesh = plsc.VectorSubcoreMesh(core_axis_name="c", subcore_axis_name="s")


def _sigmoid(v):
    return 1.0 / (1.0 + jnp.exp(-v))


@functools.partial(
    pl.kernel,
    mesh=_mesh,
    out_type=jax.ShapeDtypeStruct((BATCH, N_RANKS), jnp.float32),
    scratch_types=[
        pltpu.VMEM((PER_W,), jnp.int32),
        pltpu.VMEM((CVALS + L,), jnp.float32),
        pltpu.VMEM((CVALS + L,), jnp.float32),
        pltpu.VMEM((CVALS + L,), jnp.float32),
        pltpu.VMEM((ROWS, N_RANKS), jnp.float32),
        pltpu.VMEM((32,), jnp.float32),
        pltpu.SemaphoreType.DMA,
        pltpu.SemaphoreType.DMA,
        pltpu.SemaphoreType.DMA,
        pltpu.SemaphoreType.DMA,
        pltpu.SemaphoreType.DMA,
    ],
)
def _pbm_kernel(x_hbm, exam_hbm, rel_hbm, out_hbm,
                idx_v, vals_a, vals_b, vals_c, resv, exam_v,
                gsem_a, gsem_b, gsem_c, isem, osem):
    wid = lax.axis_index("s") * NC + lax.axis_index("c")
    base = wid * PER_W
    base_row = wid * ROWS

    vals = (vals_a, vals_b, vals_c)
    gsem = (gsem_a, gsem_b, gsem_c)

    pltpu.sync_copy(exam_hbm, exam_v)
    e0 = _sigmoid(exam_v[pl.ds(0, L)])
    e1 = _sigmoid(exam_v[pl.ds(4, L)])

    idx_copies = [
        pltpu.async_copy(
            x_hbm.at[pl.ds(base + c * CVALS, CVALS)],
            idx_v.at[pl.ds(c * CVALS, CVALS)],
            isem,
        )
        for c in range(NCH)
    ]

    def start_gather(c):
        idx_copies[c].wait()
        return pltpu.async_copy(
            rel_hbm.at[idx_v.at[pl.ds(c * CVALS, CVALS)]],
            vals[c % NBUF].at[pl.ds(0, CVALS)],
            gsem[c % NBUF],
        )

    gathers = [start_gather(c) for c in range(NBUF - 1)]
    out_copies = []
    for c in range(NCH):
        if c + NBUF - 1 < NCH:
            gathers.append(start_gather(c + NBUF - 1))
        gathers[c].wait()
        buf = vals[c % NBUF]

        def group_body(g, _, buf=buf, c=c):
            for u in range(UNROLL):
                r = g * UNROLL + u
                b = r * N_RANKS
                v0 = buf[pl.ds(b, L)]
                v1 = buf[pl.ds(b + 4, L)]
                row = c * CROWS + r
                resv[row, pl.ds(0, L)] = e0 * _sigmoid(v0)
                resv[row, pl.ds(4, L)] = e1 * _sigmoid(v1)
            return 0

        lax.fori_loop(0, CROWS // UNROLL, group_body, 0)
        out_copies.append(
            pltpu.async_copy(
                resv.at[pl.ds(c * CROWS, CROWS)],
                out_hbm.at[pl.ds(base_row + c * CROWS, CROWS)],
                osem,
            )
        )

    for oc in out_copies:
        oc.wait()


def kernel(x, exam_table, rel_table):
    xf = x.reshape(TOTAL)
    exam = jnp.pad(exam_table.reshape(N_RANKS), (0, 32 - N_RANKS))
    rel = rel_table.reshape(N_ITEMS)
    return _pbm_kernel(xf, exam, rel)
